# baseline (speedup 1.0000x reference)
import jax
import jax.numpy as jnp
from jax.experimental import pallas as pl
from jax.experimental.pallas import tpu as pltpu
from jax.experimental.pallas import tpu_sc as plsc

_GATHER_WINDOW = 128
_VOCAB_TILE = 2048


def _sc_gather(table2, ids_hi):
    n = ids_hi.shape[1]
    d2 = table2.shape[1]
    mesh = plsc.VectorSubcoreMesh(core_axis_name="c", subcore_axis_name="s")

    @pl.kernel(out_type=jax.ShapeDtypeStruct((n, d2), table2.dtype), mesh=mesh)
    def gather_kernel(table_hbm, ids_hbm, x_hbm):
        def body(i_vmem, o_vmem):
            pltpu.sync_copy(table_hbm.at[i_vmem.at[0]], o_vmem)

        pltpu.emit_pipeline(
            body,
            grid=(n // _GATHER_WINDOW,),
            in_specs=[pl.BlockSpec((1, _GATHER_WINDOW), lambda i: (0, i))],
            out_specs=[pl.BlockSpec((_GATHER_WINDOW, d2), lambda i: (i, 0))],
            core_axis_name=("c", "s"),
            dimension_semantics=(pltpu.PARALLEL,),
        )(ids_hbm, x_hbm)

    return gather_kernel(table2, ids_hi)


def _tc_project(x2, parity, W, b2):
    n, d2 = x2.shape
    d, v = W.shape

    def mm_kernel(x2_ref, p_ref, w_ref, b_ref, o_ref, xs_ref):
        @pl.when(pl.program_id(0) == 0)
        def _():
            p = p_ref[...]
            xs_ref[...] = x2_ref[:, :d] * (1.0 - p) + x2_ref[:, d:] * p

        o_ref[...] = (
            jnp.dot(xs_ref[...], w_ref[...], preferred_element_type=jnp.float32)
            + b_ref[...]
        )

    return pl.pallas_call(
        mm_kernel,
        grid=(pl.cdiv(v, _VOCAB_TILE),),
        in_specs=[
            pl.BlockSpec((n, d2), lambda i: (0, 0)),
            pl.BlockSpec((n, 1), lambda i: (0, 0)),
            pl.BlockSpec((d, _VOCAB_TILE), lambda i: (0, i)),
            pl.BlockSpec((1, _VOCAB_TILE), lambda i: (0, i)),
        ],
        out_specs=pl.BlockSpec((n, _VOCAB_TILE), lambda i: (0, i)),
        out_shape=jax.ShapeDtypeStruct((n, v), jnp.float32),
        scratch_shapes=[pltpu.VMEM((n, d), jnp.float32)],
    )(x2, parity, W, b2)


def kernel(input_ids, table, W, b):
    bsz, seq = input_ids.shape
    v, d = table.shape
    ids = input_ids.reshape(1, bsz * seq).astype(jnp.int32)
    ids_hi = ids // 2
    parity = (ids & 1).reshape(bsz * seq, 1).astype(jnp.float32)
    table2 = table.reshape(v // 2, 2 * d)
    x2 = _sc_gather(table2, ids_hi)
    logits = _tc_project(x2, parity, W, b.reshape(1, -1))
    return logits.reshape(bsz, seq, -1)


# speedup vs baseline: 1.0078x; 1.0078x over previous
import jax
import jax.numpy as jnp
from jax.experimental import pallas as pl
from jax.experimental.pallas import tpu as pltpu
from jax.experimental.pallas import tpu_sc as plsc

_GATHER_WINDOW = 128
_VOCAB_TILE = 2048


def _sc_gather(table2, ids_hi):
    n = ids_hi.shape[1]
    d2 = table2.shape[1]
    mesh = plsc.VectorSubcoreMesh(core_axis_name="c", subcore_axis_name="s")

    @pl.kernel(out_type=jax.ShapeDtypeStruct((n, d2), table2.dtype), mesh=mesh)
    def gather_kernel(table_hbm, ids_hbm, x_hbm):
        def body(i_vmem, o_vmem):
            pltpu.sync_copy(table_hbm.at[i_vmem.at[0]], o_vmem)

        pltpu.emit_pipeline(
            body,
            grid=(n // _GATHER_WINDOW,),
            in_specs=[pl.BlockSpec((1, _GATHER_WINDOW), lambda i: (0, i))],
            out_specs=[pl.BlockSpec((_GATHER_WINDOW, d2), lambda i: (i, 0))],
            core_axis_name=("c", "s"),
            dimension_semantics=(pltpu.PARALLEL,),
        )(ids_hbm, x_hbm)

    return gather_kernel(table2, ids_hi)


def _tc_project(x2, parity, W, b2):
    n, d2 = x2.shape
    d, v = W.shape

    def mm_kernel(x2_ref, p_ref, w_ref, b_ref, o_ref, xs_ref):
        @pl.when(pl.program_id(0) == 0)
        def _():
            p = p_ref[...]
            xs_ref[...] = (x2_ref[:, :d] * (1.0 - p) + x2_ref[:, d:] * p).astype(
                jnp.bfloat16
            )

        o_ref[...] = (
            jnp.dot(
                xs_ref[...],
                w_ref[...].astype(jnp.bfloat16),
                preferred_element_type=jnp.float32,
            )
            + b_ref[...]
        )

    return pl.pallas_call(
        mm_kernel,
        grid=(pl.cdiv(v, _VOCAB_TILE),),
        in_specs=[
            pl.BlockSpec((n, d2), lambda i: (0, 0)),
            pl.BlockSpec((n, 1), lambda i: (0, 0)),
            pl.BlockSpec((d, _VOCAB_TILE), lambda i: (0, i)),
            pl.BlockSpec((1, _VOCAB_TILE), lambda i: (0, i)),
        ],
        out_specs=pl.BlockSpec((n, _VOCAB_TILE), lambda i: (0, i)),
        out_shape=jax.ShapeDtypeStruct((n, v), jnp.float32),
        scratch_shapes=[pltpu.VMEM((n, d), jnp.bfloat16)],
    )(x2, parity, W, b2)


def kernel(input_ids, table, W, b):
    bsz, seq = input_ids.shape
    v, d = table.shape
    ids = input_ids.reshape(1, bsz * seq).astype(jnp.int32)
    ids_hi = ids // 2
    parity = (ids & 1).reshape(bsz * seq, 1).astype(jnp.float32)
    table2 = table.reshape(v // 2, 2 * d)
    x2 = _sc_gather(table2, ids_hi)
    logits = _tc_project(x2, parity, W, b.reshape(1, -1))
    return logits.reshape(bsz, seq, -1)


# speedup vs baseline: 1.0281x; 1.0201x over previous
import functools

import jax
import jax.numpy as jnp
from jax import lax
from jax.experimental import pallas as pl
from jax.experimental.pallas import tpu as pltpu
from jax.experimental.pallas import tpu_sc as plsc

_VOCAB_TILE = 2048


def _sc_gather(table, ids):
    n = ids.shape[0]
    d = table.shape[1]
    info = plsc.get_sparse_core_info()
    nc = info.num_cores
    half = n // nc
    mesh = plsc.ScalarSubcoreMesh(axis_name="c", num_cores=nc)

    @functools.partial(
        pl.kernel,
        mesh=mesh,
        out_type=jax.ShapeDtypeStruct((n, d), table.dtype),
        scratch_types=[
            pltpu.SMEM((half,), jnp.int32),
            pltpu.SemaphoreType.DMA,
            pltpu.SemaphoreType.DMA,
        ],
    )
    def gather_kernel(table_hbm, idx_hbm, out_hbm, idx_s, isem, sem):
        cid = lax.axis_index("c")
        base = cid * half
        pltpu.async_copy(idx_hbm.at[pl.ds(base, half)], idx_s, isem).wait()

        @pl.loop(0, half)
        def _(i):
            pltpu.async_copy(table_hbm.at[idx_s[i]], out_hbm.at[base + i], sem)

        @pl.loop(0, half)
        def _(i):
            pltpu.make_async_copy(
                table_hbm.at[idx_s[i]], out_hbm.at[base + i], sem
            ).wait()

    return gather_kernel(table, ids)


def _tc_project(x, W, b2):
    n, d = x.shape
    v = W.shape[1]

    def mm_kernel(x_ref, w_ref, b_ref, o_ref, xs_ref):
        @pl.when(pl.program_id(0) == 0)
        def _():
            xs_ref[...] = x_ref[...].astype(jnp.bfloat16)

        o_ref[...] = (
            jnp.dot(
                xs_ref[...],
                w_ref[...].astype(jnp.bfloat16),
                preferred_element_type=jnp.float32,
            )
            + b_ref[...]
        )

    return pl.pallas_call(
        mm_kernel,
        grid=(pl.cdiv(v, _VOCAB_TILE),),
        in_specs=[
            pl.BlockSpec((n, d), lambda i: (0, 0)),
            pl.BlockSpec((d, _VOCAB_TILE), lambda i: (0, i)),
            pl.BlockSpec((1, _VOCAB_TILE), lambda i: (0, i)),
        ],
        out_specs=pl.BlockSpec((n, _VOCAB_TILE), lambda i: (0, i)),
        out_shape=jax.ShapeDtypeStruct((n, v), jnp.float32),
        scratch_shapes=[pltpu.VMEM((n, d), jnp.bfloat16)],
    )(x, W, b2)


def kernel(input_ids, table, W, b):
    bsz, seq = input_ids.shape
    ids = input_ids.reshape(bsz * seq).astype(jnp.int32)
    x = _sc_gather(table, ids)
    logits = _tc_project(x, W, b.reshape(1, -1))
    return logits.reshape(bsz, seq, -1)
